# baseline (speedup 1.0000x reference)
import functools
import math

import jax
import jax.numpy as jnp
from jax import lax
from jax.experimental import pallas as pl
from jax.experimental.pallas import tpu as pltpu
from jax.experimental.pallas import tpu_sc as plsc

D_MODEL = 64
SCALE = math.sqrt(D_MODEL)
NC = 2
NS = 16
NW = NC * NS
LANES = 16


def _make_lookup(B, C):
    b_per_w = B // NW
    n_chunks = b_per_w // C
    assert b_per_w * NW == B and n_chunks * C == b_per_w

    mesh = plsc.VectorSubcoreMesh(core_axis_name="c", subcore_axis_name="s")

    @functools.partial(
        pl.kernel,
        mesh=mesh,
        out_type=jax.ShapeDtypeStruct((B, D_MODEL), jnp.float32),
        scratch_types=[
            pltpu.VMEM((C,), jnp.int32),
            pltpu.VMEM((C, D_MODEL), jnp.float32),
            pltpu.SemaphoreType.DMA,
        ],
        compiler_params=pltpu.CompilerParams(use_tc_tiling_on_sc=False),
    )
    def lookup(idx_hbm, table_hbm, out_hbm, idx_v, rows_v, sem):
        wid = lax.axis_index("s") * NC + lax.axis_index("c")
        base = wid * b_per_w

        def chunk_body(g, carry):
            off = base + g * C
            pltpu.sync_copy(idx_hbm.at[pl.ds(off, C)], idx_v)
            pltpu.async_copy(table_hbm.at[idx_v], rows_v, sem).wait()

            def scale_body(i, c2):
                for j in range(D_MODEL // LANES):
                    s = pl.ds(j * LANES, LANES)
                    rows_v[i, s] = rows_v[i, s] * SCALE
                return c2

            lax.fori_loop(0, C, scale_body, 0)
            pltpu.sync_copy(rows_v, out_hbm.at[pl.ds(off, C)])
            return carry

        lax.fori_loop(0, n_chunks, chunk_body, 0)

    return lookup


def kernel(x, table):
    R, S = x.shape
    B = R * S
    idx = x.reshape(B).astype(jnp.int32)
    out = _make_lookup(B, 128)(idx, table)
    return out.reshape(R, S, D_MODEL)


# speedup vs baseline: 1.1599x; 1.1599x over previous
import functools
import math

import jax
import jax.numpy as jnp
from jax import lax
from jax.experimental import pallas as pl
from jax.experimental.pallas import tpu as pltpu
from jax.experimental.pallas import tpu_sc as plsc

D_MODEL = 64
SCALE = math.sqrt(D_MODEL)
NC = 2
NS = 16
NW = NC * NS
LANES = 16


def _make_lookup(B, C, NB):
    b_per_w = B // NW
    n_chunks = b_per_w // C
    assert b_per_w * NW == B and n_chunks * C == b_per_w and n_chunks >= 2 * NB

    mesh = plsc.VectorSubcoreMesh(core_axis_name="c", subcore_axis_name="s")

    scratch = [
        pltpu.VMEM((n_chunks, C), jnp.int32),
        pltpu.VMEM((NB, C, D_MODEL), jnp.float32),
        pltpu.VMEM((NB, C, D_MODEL), jnp.float32),
    ] + [pltpu.SemaphoreType.DMA] * (2 * NB)

    @functools.partial(
        pl.kernel,
        mesh=mesh,
        out_type=jax.ShapeDtypeStruct((B, D_MODEL), jnp.float32),
        scratch_types=scratch,
        compiler_params=pltpu.CompilerParams(use_tc_tiling_on_sc=False),
    )
    def lookup(idx_hbm, table_hbm, out_hbm, idx_v, gbuf, obuf, *sems):
        gsem = sems[:NB]
        osem = sems[NB:]
        wid = lax.axis_index("s") * NC + lax.axis_index("c")
        base = wid * b_per_w

        pltpu.sync_copy(idx_hbm.at[wid], idx_v)

        def fire_gather(g, b):
            pltpu.async_copy(table_hbm.at[idx_v.at[g]], gbuf.at[b], gsem[b])

        def wait_gather(b):
            pltpu.make_async_copy(
                table_hbm.at[pl.ds(0, C)], gbuf.at[b], gsem[b]).wait()

        def fire_write(g, b):
            pltpu.async_copy(obuf.at[b], out_hbm.at[pl.ds(base + g * C, C)],
                             osem[b])

        def wait_write(b):
            pltpu.make_async_copy(
                obuf.at[b], out_hbm.at[pl.ds(0, C)], osem[b]).wait()

        def scale(b):
            src = gbuf.at[b]
            dst = obuf.at[b]

            @pl.loop(0, C, unroll=4)
            def _(i):
                for j in range(D_MODEL // LANES):
                    s = pl.ds(j * LANES, LANES)
                    dst[i, s] = src[i, s] * SCALE

        for b in range(NB):
            fire_gather(b, b)

        for b in range(NB):
            wait_gather(b)
            scale(b)
            fire_gather(b + NB, b)
            fire_write(b, b)

        @pl.loop(NB, n_chunks - NB, step=NB)
        def _(G):
            for b in range(NB):
                g = G + b
                wait_gather(b)
                wait_write(b)
                scale(b)
                fire_gather(g + NB, b)
                fire_write(g, b)

        for b in range(NB):
            wait_gather(b)
            wait_write(b)
            scale(b)
            fire_write(n_chunks - NB + b, b)
        for b in range(NB):
            wait_write(b)

    return lookup


def kernel(x, table):
    R, S = x.shape
    B = R * S
    C = 128
    idx = x.reshape(NW, (B // NW) // C, C).astype(jnp.int32)
    out = _make_lookup(B, C, 4)(idx, table)
    return out.reshape(R, S, D_MODEL)
